# baseline (speedup 1.0000x reference)
import functools

import jax
import jax.numpy as jnp
from jax import lax
from jax.experimental import pallas as pl
from jax.experimental.pallas import tpu as pltpu
from jax.experimental.pallas import tpu_sc as plsc

_NC, _NS, _L = 2, 16, 16
_NW = _NC * _NS
_SL = 8
_U = 8


def _emb_body(R, NT, ICH, x_hbm, tab_hbm, out_hbm,
              in0, in1, out0, out1, tab_v, sin0, sin1, sout0, sout1):
    wid = lax.axis_index("s") * _NC + lax.axis_index("c")
    i0 = wid * ICH
    pltpu.sync_copy(tab_hbm, tab_v)

    def in_slice(t):
        return x_hbm.at[pl.ds(t * _SL, _SL), pl.ds(i0, ICH)]

    def out_slice(t):
        return out_hbm.at[pl.ds(t * _SL, _SL), :, pl.ds(i0, ICH)]

    def compute(in_v, out_v):
        for s in range(_SL):

            @plsc.parallel_loop(0, ICH // _L, unroll=_U)
            def _grp(g, s=s):
                iv = in_v[s, pl.ds(g * _L, _L)] * R
                for q in range(R):
                    val = plsc.load_gather(tab_v, [iv + q])
                    out_v[s, q, pl.ds(g * _L, _L)] = val

    pltpu.async_copy(in_slice(0), in0, sin0)

    def pair(p, carry):
        t0 = p * 2

        pltpu.make_async_copy(in_slice(t0), in0, sin0).wait()

        @pl.when(t0 + 1 < NT)
        def _():
            pltpu.async_copy(in_slice(t0 + 1), in1, sin1)

        @pl.when(t0 >= 2)
        def _():
            pltpu.make_async_copy(out0, out_slice(t0 - 2), sout0).wait()

        compute(in0, out0)
        pltpu.async_copy(out0, out_slice(t0), sout0)

        @pl.when(t0 + 1 < NT)
        def _():
            t1 = t0 + 1
            pltpu.make_async_copy(in_slice(t1), in1, sin1).wait()

            @pl.when(t1 + 1 < NT)
            def _():
                pltpu.async_copy(in_slice(t1 + 1), in0, sin0)

            @pl.when(t1 >= 2)
            def _():
                pltpu.make_async_copy(out1, out_slice(t1 - 2), sout1).wait()

            compute(in1, out1)
            pltpu.async_copy(out1, out_slice(t1), sout1)

        return carry

    lax.fori_loop(0, (NT + 1) // 2, pair, 0)

    last_even = ((NT - 1) // 2) * 2
    pltpu.make_async_copy(out0, out_slice(last_even), sout0).wait()
    if NT > 1:
        last_odd = ((NT - 2) // 2) * 2 + 1
        pltpu.make_async_copy(out1, out_slice(last_odd), sout1).wait()


def kernel(x, table):
    B, S = x.shape
    V, R = table.shape
    ICH = B // _NW
    NT = S // _SL

    body = functools.partial(_emb_body, R, NT, ICH)
    k = pl.kernel(
        body,
        out_type=jax.ShapeDtypeStruct((S, R, B), table.dtype),
        compiler_params=pltpu.CompilerParams(
            needs_layout_passes=False, use_tc_tiling_on_sc=True),
        mesh=plsc.VectorSubcoreMesh(
            core_axis_name="c", subcore_axis_name="s",
            num_cores=_NC, num_subcores=_NS,
        ),
        scratch_types=[
            pltpu.VMEM((_SL, ICH), jnp.int32),
            pltpu.VMEM((_SL, ICH), jnp.int32),
            pltpu.VMEM((_SL, R, ICH), jnp.float32),
            pltpu.VMEM((_SL, R, ICH), jnp.float32),
            pltpu.VMEM((V * R,), jnp.float32),
            pltpu.SemaphoreType.DMA,
            pltpu.SemaphoreType.DMA,
            pltpu.SemaphoreType.DMA,
            pltpu.SemaphoreType.DMA,
        ],
    )
    ot = k(jnp.transpose(x), table.reshape(V * R))
    return jnp.transpose(ot, (2, 0, 1))


# speedup vs baseline: 1.0100x; 1.0100x over previous
import functools

import jax
import jax.numpy as jnp
from jax import lax
from jax.experimental import pallas as pl
from jax.experimental.pallas import tpu as pltpu
from jax.experimental.pallas import tpu_sc as plsc

_NC, _NS, _L = 2, 16, 16
_NW = _NC * _NS
_SL = 8
_U = 8


def _emb_body(R, NT, ICH, x_hbm, tab_hbm, out_hbm,
              in0, in1, out0, out1, tab_v, sin0, sin1, sout0, sout1):
    wid = lax.axis_index("s") * _NC + lax.axis_index("c")
    i0 = wid * ICH

    def in_slice(t):
        return x_hbm.at[pl.ds(t * _SL, _SL), pl.ds(i0, ICH)]

    def out_slice(t):
        return out_hbm.at[pl.ds(t * _SL, _SL), :, pl.ds(i0, ICH)]

    def compute(in_v, out_v):
        for s in range(_SL):

            @plsc.parallel_loop(0, ICH // _L, unroll=_U)
            def _grp(g, s=s):
                iv = in_v[s, pl.ds(g * _L, _L)] * R
                for q in range(R):
                    val = plsc.load_gather(tab_v, [iv + q])
                    out_v[s, q, pl.ds(g * _L, _L)] = val

    pltpu.async_copy(in_slice(0), in0, sin0)
    pltpu.sync_copy(tab_hbm, tab_v)

    def pair(p, carry):
        t0 = p * 2

        pltpu.make_async_copy(in_slice(t0), in0, sin0).wait()

        @pl.when(t0 + 1 < NT)
        def _():
            pltpu.async_copy(in_slice(t0 + 1), in1, sin1)

        @pl.when(t0 >= 2)
        def _():
            pltpu.make_async_copy(out0, out_slice(t0 - 2), sout0).wait()

        compute(in0, out0)
        pltpu.async_copy(out0, out_slice(t0), sout0)

        @pl.when(t0 + 1 < NT)
        def _():
            t1 = t0 + 1
            pltpu.make_async_copy(in_slice(t1), in1, sin1).wait()

            @pl.when(t1 + 1 < NT)
            def _():
                pltpu.async_copy(in_slice(t1 + 1), in0, sin0)

            @pl.when(t1 >= 2)
            def _():
                pltpu.make_async_copy(out1, out_slice(t1 - 2), sout1).wait()

            compute(in1, out1)
            pltpu.async_copy(out1, out_slice(t1), sout1)

        return carry

    lax.fori_loop(0, (NT + 1) // 2, pair, 0)

    last_even = ((NT - 1) // 2) * 2
    pltpu.make_async_copy(out0, out_slice(last_even), sout0).wait()
    if NT > 1:
        last_odd = ((NT - 2) // 2) * 2 + 1
        pltpu.make_async_copy(out1, out_slice(last_odd), sout1).wait()


def kernel(x, table):
    B, S = x.shape
    V, R = table.shape
    ICH = B // _NW
    NT = S // _SL

    body = functools.partial(_emb_body, R, NT, ICH)
    k = pl.kernel(
        body,
        out_type=jax.ShapeDtypeStruct((S, R, B), table.dtype),
        compiler_params=pltpu.CompilerParams(
            needs_layout_passes=False, use_tc_tiling_on_sc=True),
        mesh=plsc.VectorSubcoreMesh(
            core_axis_name="c", subcore_axis_name="s",
            num_cores=_NC, num_subcores=_NS,
        ),
        scratch_types=[
            pltpu.VMEM((_SL, ICH), jnp.int32),
            pltpu.VMEM((_SL, ICH), jnp.int32),
            pltpu.VMEM((_SL, R, ICH), jnp.float32),
            pltpu.VMEM((_SL, R, ICH), jnp.float32),
            pltpu.VMEM((V * R,), jnp.float32),
            pltpu.SemaphoreType.DMA,
            pltpu.SemaphoreType.DMA,
            pltpu.SemaphoreType.DMA,
            pltpu.SemaphoreType.DMA,
        ],
    )
    ot = k(jnp.transpose(x), table.reshape(V * R))
    return jnp.transpose(ot, (2, 0, 1))
